# baseline (speedup 1.0000x reference)
import jax
import jax.numpy as jnp
from jax.experimental import pallas as pl
from jax.experimental.pallas import tpu as pltpu

_PAD = 1
_T = 256


def _body(tok_ref, w_ref, wnext_ref, pad_ref, out_ref):
    w_ext = jnp.concatenate([w_ref[...], wnext_ref[...]], axis=0)
    w = pltpu.roll(w_ext, _T + 6, 0)[:_T, :]
    pad = jnp.broadcast_to(pad_ref[_PAD : _PAD + 1, :], w.shape)
    bsz = out_ref.shape[0]
    for b in range(bsz):
        mask = tok_ref[:, b : b + 1] != _PAD
        out_ref[b] = jnp.where(mask, w, pad)


def kernel(input, weights):
    bsz, seq_len = input.shape
    dim = weights.shape[1]
    tok_t = input.T
    grid = (seq_len // _T,)
    nb = _T // 8
    return pl.pallas_call(
        _body,
        grid=grid,
        in_specs=[
            pl.BlockSpec((_T, bsz), lambda j: (j, 0)),
            pl.BlockSpec((_T, dim), lambda j: (j, 0)),
            pl.BlockSpec((8, dim), lambda j: (j * nb + nb, 0)),
            pl.BlockSpec((8, dim), lambda j: (0, 0)),
        ],
        out_specs=pl.BlockSpec((bsz, _T, dim), lambda j: (0, j, 0)),
        out_shape=jax.ShapeDtypeStruct((bsz, seq_len, dim), weights.dtype),
    )(tok_t, weights, weights, weights)
